# baseline (speedup 1.0000x reference)
import functools

import jax
import jax.numpy as jnp
from jax import lax
from jax.experimental import pallas as pl
from jax.experimental.pallas import tpu as pltpu
from jax.experimental.pallas import tpu_sc as plsc

_SEQ = 8192
_C = 13
_B = 4
_LANES = 128
_NBLK = _SEQ // _LANES
_NUM_CORES = 2
_NUM_SUBCORES = 16
_NW = _NUM_CORES * _NUM_SUBCORES
_HALF = _SEQ // 2
_HALF_BLK = _NBLK // 2
_QTR = _HALF // 2
_QTR_BLK = _HALF_BLK // 2


@functools.partial(
    pl.kernel,
    mesh=plsc.VectorSubcoreMesh(core_axis_name="c", subcore_axis_name="s"),
    out_type=jax.ShapeDtypeStruct((_C, _NBLK, _B, _LANES), jnp.float32),
    compiler_params=pltpu.CompilerParams(needs_layout_passes=False),
    scratch_types=[
        pltpu.VMEM((2, _QTR_BLK, _LANES), jnp.float32),
        pltpu.SemaphoreType.DMA,
        pltpu.SemaphoreType.DMA,
        pltpu.SemaphoreType.DMA,
    ],
)
def _bcast_sc(wt_hbm, out_hbm, staged, sem_s0, sem_s1, sem_out):
    wid = lax.axis_index("s") * _NUM_CORES + lax.axis_index("c")
    c = wid // 2
    h = wid % 2

    @pl.when(c < _C)
    def _():
        stage_sems = (sem_s0, sem_s1)
        stages = []
        for q in range(2):
            st = pltpu.make_async_copy(
                wt_hbm.at[pl.ds(c, 1), pl.ds(h * _HALF + q * _QTR, _QTR)],
                staged.at[q].reshape(1, _QTR),
                stage_sems[q],
            )
            st.start()
            stages.append(st)
        copies = []
        for q in range(2):
            stages[q].wait()
            for b in range(_B):
                cp = pltpu.make_async_copy(
                    staged.at[q],
                    out_hbm.at[
                        c, pl.ds(h * _HALF_BLK + q * _QTR_BLK, _QTR_BLK), b, :
                    ],
                    sem_out,
                )
                cp.start()
                copies.append(cp)
        for cp in copies:
            cp.wait()


def kernel(x, row_embed_weight):
    del x
    wt = row_embed_weight.T
    tmp = _bcast_sc(wt)
    return jnp.transpose(tmp, (2, 0, 1, 3)).reshape(_B, _C, _SEQ)


# speedup vs baseline: 1.0611x; 1.0611x over previous
import functools

import jax
import jax.numpy as jnp
from jax import lax
from jax.experimental import pallas as pl
from jax.experimental.pallas import tpu as pltpu
from jax.experimental.pallas import tpu_sc as plsc

_SEQ = 8192
_C = 13
_B = 4
_LANES = 128
_NBLK = _SEQ // _LANES
_NUM_CORES = 2
_NUM_SUBCORES = 16
_NW = _NUM_CORES * _NUM_SUBCORES
_HALF = _SEQ // 2
_HALF_BLK = _NBLK // 2
_QTR = _HALF // 2
_QTR_BLK = _HALF_BLK // 2


@functools.partial(
    pl.kernel,
    mesh=plsc.VectorSubcoreMesh(core_axis_name="c", subcore_axis_name="s"),
    out_type=jax.ShapeDtypeStruct((_C, _NBLK, _B, _LANES), jnp.float32),
    compiler_params=pltpu.CompilerParams(needs_layout_passes=False),
    scratch_types=[
        pltpu.VMEM((_HALF_BLK, _LANES), jnp.float32),
        pltpu.SemaphoreType.DMA,
    ],
)
def _bcast_sc(wt_hbm, out_hbm, staged, sem):
    wid = lax.axis_index("s") * _NUM_CORES + lax.axis_index("c")
    c = wid // 2
    h = wid % 2

    @pl.when(c < _C)
    def _():
        pltpu.sync_copy(
            wt_hbm.at[pl.ds(c, 1), pl.ds(h * _HALF, _HALF)],
            staged.reshape(1, _HALF),
        )
        copies = []
        for b in range(_B):
            cp = pltpu.make_async_copy(
                staged,
                out_hbm.at[c, pl.ds(h * _HALF_BLK, _HALF_BLK), b, :],
                sem,
            )
            cp.start()
            copies.append(cp)
        for cp in copies:
            cp.wait()


@functools.partial(
    pl.kernel,
    mesh=plsc.ScalarSubcoreMesh(axis_name="core", num_cores=2),
    out_type=jax.ShapeDtypeStruct((_C, _NBLK, _B, _LANES), jnp.float32),
    compiler_params=pltpu.CompilerParams(needs_layout_passes=False),
    scratch_types=[
        pltpu.VMEM_SHARED((_C, _HALF_BLK, _LANES), jnp.float32),
        pltpu.SemaphoreType.DMA,
        pltpu.SemaphoreType.DMA,
    ],
)
def _bcast_scs(wt_hbm, out_hbm, staged, sem_in, sem_out):
    h = lax.axis_index("core")
    stages = []
    for c in range(_C):
        st = pltpu.make_async_copy(
            wt_hbm.at[pl.ds(c, 1), pl.ds(h * _HALF, _HALF)],
            staged.at[c].reshape(1, _HALF),
            sem_in,
        )
        st.start()
        stages.append(st)
    copies = []
    for c in range(_C):
        stages[c].wait()
        for b in range(_B):
            cp = pltpu.make_async_copy(
                staged.at[c],
                out_hbm.at[c, pl.ds(h * _HALF_BLK, _HALF_BLK), b, :],
                sem_out,
            )
            cp.start()
            copies.append(cp)
    for cp in copies:
        cp.wait()


def kernel(x, row_embed_weight):
    del x
    wt = row_embed_weight.T
    tmp = _bcast_scs(wt)
    return jnp.transpose(tmp, (2, 0, 1, 3)).reshape(_B, _C, _SEQ)


# speedup vs baseline: 1.0640x; 1.0027x over previous
import functools

import jax
import jax.numpy as jnp
from jax import lax
from jax.experimental import pallas as pl
from jax.experimental.pallas import tpu as pltpu
from jax.experimental.pallas import tpu_sc as plsc

_SEQ = 8192
_C = 13
_B = 4
_LANES = 128
_NBLK = _SEQ // _LANES
_NUM_CORES = 2
_HALF = _SEQ // 2
_HALF_BLK = _NBLK // 2


@functools.partial(
    pl.kernel,
    mesh=plsc.ScalarSubcoreMesh(axis_name="core", num_cores=_NUM_CORES),
    out_type=jax.ShapeDtypeStruct((_C, _NBLK, _B, _LANES), jnp.float32),
    compiler_params=pltpu.CompilerParams(needs_layout_passes=False),
    scratch_types=[
        pltpu.VMEM_SHARED((_C, _HALF_BLK, _LANES), jnp.float32),
        pltpu.SemaphoreType.DMA,
        pltpu.SemaphoreType.DMA,
    ],
)
def _bcast_scs(wt_hbm, out_hbm, staged, sem_in, sem_out):
    h = lax.axis_index("core")
    stages = []
    for c in range(_C):
        st = pltpu.make_async_copy(
            wt_hbm.at[pl.ds(c, 1), pl.ds(h * _HALF, _HALF)],
            staged.at[c].reshape(1, _HALF),
            sem_in,
        )
        st.start()
        stages.append(st)
    copies = []
    for c in range(_C):
        stages[c].wait()
        for b in range(_B):
            cp = pltpu.make_async_copy(
                staged.at[c],
                out_hbm.at[c, pl.ds(h * _HALF_BLK, _HALF_BLK), b, :],
                sem_out,
            )
            cp.start()
            copies.append(cp)
    for cp in copies:
        cp.wait()


def kernel(x, row_embed_weight):
    del x
    wt = row_embed_weight.T
    tmp = _bcast_scs(wt)
    return jnp.transpose(tmp, (2, 0, 1, 3)).reshape(_B, _C, _SEQ)
